# Initial kernel scaffold; baseline (speedup 1.0000x reference)
import jax
import jax.numpy as jnp
from jax.experimental import pallas as pl


def kernel(x, edge_index_l1_r0, edge_index_l1_r1, edge_index_l2_r0, edge_index_l2_r1, edge_index_l3_r0, edge_index_l3_r1, W1_r0, b1_r0, W1_r1, b1_r1, W2_r0, b2_r0, W2_r1, b2_r1, W3_r0, b3_r0, W3_r1, b3_r1):
    raise NotImplementedError("write your pallas kernel here")


# speedup vs baseline: 2.1155x; 2.1155x over previous
import functools

import jax
import jax.numpy as jnp
from jax import lax
from jax.experimental import pallas as pl
from jax.experimental.pallas import tpu as pltpu
from jax.experimental.pallas import tpu_sc as plsc

N = 10000
D = 128
E = 160000

DA = 144
CHUNK = 128
N_TILES = 16
N_CHUNKS = 79
EDGES_PER_TILE = N_CHUNKS * CHUNK
E_PAD = N_TILES * EDGES_PER_TILE
CAP = EDGES_PER_TILE + CHUNK
N_PAD = 10240
HALF = 3584
N_PASS = 3
ACC_ROWS = HALF + CHUNK
DUMMY_DST = N


def _build_sc_seg_sum():
    mesh = plsc.VectorSubcoreMesh(core_axis_name="c", subcore_axis_name="s")

    @functools.partial(
        pl.kernel,
        out_type=jax.ShapeDtypeStruct((2, N_PAD, DA), jnp.float32),
        mesh=mesh,
        compiler_params=pltpu.CompilerParams(
            needs_layout_passes=False, use_tc_tiling_on_sc=False),
        scratch_types=(
            pltpu.VMEM((N_PASS * CAP + 16,), jnp.int32),
            pltpu.VMEM((N_PASS * CAP + 16,), jnp.int32),
            pltpu.VMEM((CHUNK,), jnp.int32),
            pltpu.VMEM((CHUNK,), jnp.int32),
            pltpu.VMEM((CHUNK,), jnp.int32),
            pltpu.VMEM((CHUNK, DA), jnp.float32),
            pltpu.VMEM_SHARED((ACC_ROWS, DA), jnp.float32),
            pltpu.SemaphoreType.DMA,
        ),
    )
    def k(table_h, src_h, dst_h, acc_h,
          bsrc_v, bdst_v, src_c, dst_c, loc_v, rows_v, acc_sh, sem):
        cid = lax.axis_index("c")
        sid = lax.axis_index("s")

        i16 = lax.iota(jnp.int32, 16)
        zero16i = jnp.zeros((16,), jnp.int32)
        dummy16 = HALF + i16 * 8
        trash = N_PASS * CAP

        def comp_body(j, cnt):
            pltpu.sync_copy(src_h.at[cid, sid, j], src_c)
            pltpu.sync_copy(dst_h.at[cid, sid, j], dst_c)
            for q in range(CHUNK // 16):
                s16 = src_c[pl.ds(q * 16, 16)]
                d16 = dst_c[pl.ds(q * 16, 16)]
                new = []
                for p in range(N_PASS):
                    if p == 0:
                        m = d16 < HALF
                    elif p == 1:
                        m = (d16 >= HALF) & (d16 < 2 * HALF)
                    else:
                        m = d16 >= 2 * HALF
                    mi = m.astype(jnp.int32)
                    cs = plsc.cumsum(mi)
                    pos = cs - mi
                    tgt = jnp.where(m, p * CAP + cnt[p] + pos, trash + i16)
                    plsc.store_scatter(bsrc_v, [tgt], s16)
                    plsc.store_scatter(bdst_v, [tgt], d16 - p * HALF)
                    new.append(cnt[p] + cs[15])
                cnt = tuple(new)
            return cnt

        cnt = lax.fori_loop(0, N_CHUNKS, comp_body, (0, 0, 0))

        for p in range(N_PASS):
            for kk in range(CHUNK // 16):
                tgt = p * CAP + cnt[p] + kk * 16 + i16
                plsc.store_scatter(bsrc_v, [tgt], zero16i)
                plsc.store_scatter(bdst_v, [tgt], dummy16)

        zero16f = jnp.zeros((16,), jnp.float32)
        for p in range(N_PASS):
            real_rows = min(HALF, N_PAD - p * HALF)
            per_tile = real_rows // N_TILES
            zoff = sid * per_tile
            gbase = p * HALF + zoff

            def zfill(i, _):
                for q in range(DA // 16):
                    rows_v[i, pl.ds(q * 16, 16)] = zero16f
                return 0

            lax.fori_loop(0, CHUNK, zfill, 0)
            for off, sz in ((0, CHUNK), (CHUNK, per_tile - CHUNK)):
                pltpu.sync_copy(rows_v.at[pl.ds(0, sz)],
                                acc_sh.at[pl.ds(zoff + off, sz)])
            plsc.subcore_barrier()

            n_chunks = (cnt[p] + CHUNK - 1) // CHUNK

            def body(j, _):
                base = p * CAP + j * CHUNK
                for q in range(CHUNK // 16):
                    loc_v[pl.ds(q * 16, 16)] = bdst_v[pl.ds(base + q * 16, 16)]
                pltpu.async_copy(table_h.at[bsrc_v.at[pl.ds(base, CHUNK)]],
                                 rows_v, sem).wait()
                pltpu.sync_copy(rows_v, acc_sh.at[loc_v], add=True)
                return 0

            lax.fori_loop(0, n_chunks, body, 0)
            plsc.subcore_barrier()

            for off, sz in ((0, CHUNK), (CHUNK, per_tile - CHUNK)):
                pltpu.sync_copy(acc_sh.at[pl.ds(zoff + off, sz)],
                                rows_v.at[pl.ds(0, sz)])
                pltpu.sync_copy(rows_v.at[pl.ds(0, sz)],
                                acc_h.at[cid, pl.ds(gbase + off, sz)])

    return k


_SC_SEG_SUM = _build_sc_seg_sum()

_BLK = 400


def _tc_combine_body(aug_out, a0, a1, w0, w1, b, pre_ref, post_ref):
    acc0 = a0[...][0]
    acc1 = a1[...][0]
    n0 = 1.0 / jnp.maximum(acc0[:, D:D + 1], 1.0)
    n1 = 1.0 / jnp.maximum(acc1[:, D:D + 1], 1.0)
    h0 = acc0[:, :D] * n0
    h1 = acc1[:, :D] * n1
    pre = (jnp.dot(h0, w0[...], preferred_element_type=jnp.float32)
           + jnp.dot(h1, w1[...], preferred_element_type=jnp.float32)
           + b[...])
    pre_ref[...] = pre
    post = jnp.maximum(pre, 0.0)
    if aug_out:
        pad = jnp.concatenate(
            [jnp.ones((_BLK, 1), jnp.float32),
             jnp.zeros((_BLK, DA - D - 1), jnp.float32)], axis=1)
        post_ref[...] = jnp.concatenate([post, pad], axis=1)
    else:
        post_ref[...] = post


def _tc_combine(acc, W0, W1, bsum, aug_out):
    return pl.pallas_call(
        functools.partial(_tc_combine_body, aug_out),
        grid=(N // _BLK,),
        in_specs=[
            pl.BlockSpec((1, _BLK, DA), lambda i: (0, i, 0)),
            pl.BlockSpec((1, _BLK, DA), lambda i: (1, i, 0)),
            pl.BlockSpec((D, D), lambda i: (0, 0)),
            pl.BlockSpec((D, D), lambda i: (0, 0)),
            pl.BlockSpec((1, D), lambda i: (0, 0)),
        ],
        out_specs=[pl.BlockSpec((_BLK, D), lambda i: (i, 0)),
                   pl.BlockSpec((_BLK, DA if aug_out else D),
                                lambda i: (i, 0))],
        out_shape=[jax.ShapeDtypeStruct((N, D), jnp.float32),
                   jax.ShapeDtypeStruct((N, DA if aug_out else D),
                                        jnp.float32)],
    )(acc, acc, W0, W1, bsum)


def _prep_edges(ei0, ei1):
    pad = E_PAD - E
    src = jnp.concatenate(
        [jnp.stack([ei0[0], ei1[0]]), jnp.zeros((2, pad), jnp.int32)], axis=1)
    dst = jnp.concatenate(
        [jnp.stack([ei0[1], ei1[1]]), jnp.full((2, pad), DUMMY_DST, jnp.int32)],
        axis=1)
    return (src.reshape(2, N_TILES, N_CHUNKS, CHUNK),
            dst.reshape(2, N_TILES, N_CHUNKS, CHUNK))


def kernel(x,
           edge_index_l1_r0, edge_index_l1_r1,
           edge_index_l2_r0, edge_index_l2_r1,
           edge_index_l3_r0, edge_index_l3_r1,
           W1_r0, b1_r0, W1_r1, b1_r1,
           W2_r0, b2_r0, W2_r1, b2_r1,
           W3_r0, b3_r0, W3_r1, b3_r1):
    s1, d1 = _prep_edges(edge_index_l1_r0, edge_index_l1_r1)
    s2, d2 = _prep_edges(edge_index_l2_r0, edge_index_l2_r1)
    s3, d3 = _prep_edges(edge_index_l3_r0, edge_index_l3_r1)

    x_aug = jnp.concatenate(
        [x, jnp.ones((N, 1), jnp.float32), jnp.zeros((N, DA - D - 1),
                                                     jnp.float32)], axis=1)

    acc = _SC_SEG_SUM(x_aug, s1, d1)
    _, h1 = _tc_combine(acc, W1_r0, W1_r1, (b1_r0 + b1_r1).reshape(1, D), True)
    acc = _SC_SEG_SUM(h1, s2, d2)
    x1, h2 = _tc_combine(acc, W2_r0, W2_r1, (b2_r0 + b2_r1).reshape(1, D), True)
    acc = _SC_SEG_SUM(h2, s3, d3)
    _, h3 = _tc_combine(acc, W3_r0, W3_r1, (b3_r0 + b3_r1).reshape(1, D), False)
    return h3, x1


# speedup vs baseline: 2.2750x; 1.0754x over previous
import functools

import jax
import jax.numpy as jnp
from jax import lax
from jax.experimental import pallas as pl
from jax.experimental.pallas import tpu as pltpu
from jax.experimental.pallas import tpu_sc as plsc

N = 10000
D = 128
E = 160000

DA = 144
CHUNK = 128
N_TILES = 16
N_CHUNKS = 79
EDGES_PER_TILE = N_CHUNKS * CHUNK
E_PAD = N_TILES * EDGES_PER_TILE
CAP = EDGES_PER_TILE + CHUNK
CAP_ROWS = CAP // CHUNK
N_PAD = 10240
HALF = 3456
N_PASS = 3
ACC_ROWS = HALF + 16
DUMMY_DST = N


def _build_sc_seg_sum():
    mesh = plsc.VectorSubcoreMesh(core_axis_name="c", subcore_axis_name="s")

    @functools.partial(
        pl.kernel,
        out_type=jax.ShapeDtypeStruct((2, N_PAD, DA), jnp.float32),
        mesh=mesh,
        compiler_params=pltpu.CompilerParams(
            needs_layout_passes=False, use_tc_tiling_on_sc=False),
        scratch_types=(
            pltpu.VMEM((N_PASS * CAP + 16,), jnp.int32),
            pltpu.VMEM((N_PASS * CAP_ROWS + 1, CHUNK), jnp.int32),
            pltpu.VMEM((CHUNK,), jnp.int32),
            pltpu.VMEM((CHUNK,), jnp.int32),
            pltpu.VMEM((CHUNK, DA), jnp.float32),
            pltpu.VMEM((CHUNK, DA), jnp.float32),
            pltpu.VMEM_SHARED((ACC_ROWS, DA), jnp.float32),
            pltpu.SemaphoreType.DMA,
            pltpu.SemaphoreType.DMA,
        ),
    )
    def k(table_h, src_h, dst_h, acc_h,
          bsrc_v, bdst_v, src_c, dst_c, rows_a, rows_b, acc_sh, sem_a, sem_b):
        cid = lax.axis_index("c")
        sid = lax.axis_index("s")

        i16 = lax.iota(jnp.int32, 16)
        zero16i = jnp.zeros((16,), jnp.int32)
        dummy16 = HALF + i16
        trash_src = N_PASS * CAP
        trash_row = N_PASS * CAP_ROWS

        def comp_body(j, cnt):
            pltpu.sync_copy(src_h.at[cid, sid, j], src_c)
            pltpu.sync_copy(dst_h.at[cid, sid, j], dst_c)
            for q in range(CHUNK // 16):
                s16 = src_c[pl.ds(q * 16, 16)]
                d16 = dst_c[pl.ds(q * 16, 16)]
                new = []
                for p in range(N_PASS):
                    if p == 0:
                        m = d16 < HALF
                    elif p == 1:
                        m = (d16 >= HALF) & (d16 < 2 * HALF)
                    else:
                        m = d16 >= 2 * HALF
                    mi = m.astype(jnp.int32)
                    cs = plsc.cumsum(mi)
                    tgt = cnt[p] + cs - mi
                    plsc.store_scatter(
                        bsrc_v,
                        [jnp.where(m, p * CAP + tgt, trash_src + i16)], s16)
                    plsc.store_scatter(
                        bdst_v,
                        [jnp.where(m, p * CAP_ROWS + (tgt >> 7), trash_row),
                         jnp.where(m, tgt & (CHUNK - 1), i16)],
                        d16 - p * HALF)
                    new.append(cnt[p] + cs[15])
                cnt = tuple(new)
            return cnt

        cnt = lax.fori_loop(0, N_CHUNKS, comp_body, (0, 0, 0))

        for p in range(N_PASS):
            for kk in range(CHUNK // 16):
                tgt = cnt[p] + kk * 16 + i16
                plsc.store_scatter(bsrc_v, [p * CAP + tgt], zero16i)
                plsc.store_scatter(
                    bdst_v,
                    [p * CAP_ROWS + (tgt >> 7), tgt & (CHUNK - 1)], dummy16)

        zero16f = jnp.zeros((16,), jnp.float32)
        bufs = (rows_a, rows_b)
        sems = (sem_a, sem_b)
        for p in range(N_PASS):
            real_rows = min(HALF, N_PAD - p * HALF)
            per_tile = real_rows // N_TILES
            zoff = sid * per_tile
            gbase = p * HALF + zoff

            def zfill(i, _):
                for q in range(DA // 16):
                    rows_a[i, pl.ds(q * 16, 16)] = zero16f
                return 0

            lax.fori_loop(0, CHUNK, zfill, 0)
            for off, sz in ((0, CHUNK), (CHUNK, per_tile - CHUNK)):
                pltpu.sync_copy(rows_a.at[pl.ds(0, sz)],
                                acc_sh.at[pl.ds(zoff + off, sz)])
            plsc.subcore_barrier()

            n_chunks = jnp.maximum((cnt[p] + CHUNK - 1) >> 7, 1)

            def gsrc(j):
                return table_h.at[bsrc_v.at[pl.ds(p * CAP + j * CHUNK, CHUNK)]]

            pltpu.async_copy(gsrc(0), bufs[0], sems[0])

            def body(jj, _):
                for b in (0, 1):
                    j = jj * 2 + b

                    @pl.when(j < n_chunks)
                    def _():
                        pltpu.make_async_copy(gsrc(j), bufs[b], sems[b]).wait()

                        @pl.when(j + 1 < n_chunks)
                        def _():
                            pltpu.async_copy(gsrc(j + 1), bufs[1 - b],
                                             sems[1 - b])

                        pltpu.sync_copy(bufs[b],
                                        acc_sh.at[bdst_v.at[p * CAP_ROWS + j]],
                                        add=True)
                return 0

            lax.fori_loop(0, (n_chunks + 1) >> 1, body, 0)
            plsc.subcore_barrier()

            for off, sz in ((0, CHUNK), (CHUNK, per_tile - CHUNK)):
                pltpu.sync_copy(acc_sh.at[pl.ds(zoff + off, sz)],
                                rows_a.at[pl.ds(0, sz)])
                pltpu.sync_copy(rows_a.at[pl.ds(0, sz)],
                                acc_h.at[cid, pl.ds(gbase + off, sz)])

    return k


_SC_SEG_SUM = _build_sc_seg_sum()

_BLK = 400


def _tc_combine_body(aug_out, a0, a1, w0, w1, b, pre_ref, post_ref):
    acc0 = a0[...][0]
    acc1 = a1[...][0]
    n0 = 1.0 / jnp.maximum(acc0[:, D:D + 1], 1.0)
    n1 = 1.0 / jnp.maximum(acc1[:, D:D + 1], 1.0)
    h0 = acc0[:, :D] * n0
    h1 = acc1[:, :D] * n1
    pre = (jnp.dot(h0, w0[...], preferred_element_type=jnp.float32)
           + jnp.dot(h1, w1[...], preferred_element_type=jnp.float32)
           + b[...])
    pre_ref[...] = pre
    post = jnp.maximum(pre, 0.0)
    if aug_out:
        pad = jnp.concatenate(
            [jnp.ones((_BLK, 1), jnp.float32),
             jnp.zeros((_BLK, DA - D - 1), jnp.float32)], axis=1)
        post_ref[...] = jnp.concatenate([post, pad], axis=1)
    else:
        post_ref[...] = post


def _tc_combine(acc, W0, W1, bsum, aug_out):
    return pl.pallas_call(
        functools.partial(_tc_combine_body, aug_out),
        grid=(N // _BLK,),
        in_specs=[
            pl.BlockSpec((1, _BLK, DA), lambda i: (0, i, 0)),
            pl.BlockSpec((1, _BLK, DA), lambda i: (1, i, 0)),
            pl.BlockSpec((D, D), lambda i: (0, 0)),
            pl.BlockSpec((D, D), lambda i: (0, 0)),
            pl.BlockSpec((1, D), lambda i: (0, 0)),
        ],
        out_specs=[pl.BlockSpec((_BLK, D), lambda i: (i, 0)),
                   pl.BlockSpec((_BLK, DA if aug_out else D),
                                lambda i: (i, 0))],
        out_shape=[jax.ShapeDtypeStruct((N, D), jnp.float32),
                   jax.ShapeDtypeStruct((N, DA if aug_out else D),
                                        jnp.float32)],
    )(acc, acc, W0, W1, bsum)


def _prep_edges(ei0, ei1):
    pad = E_PAD - E
    src = jnp.concatenate(
        [jnp.stack([ei0[0], ei1[0]]), jnp.zeros((2, pad), jnp.int32)], axis=1)
    dst = jnp.concatenate(
        [jnp.stack([ei0[1], ei1[1]]), jnp.full((2, pad), DUMMY_DST, jnp.int32)],
        axis=1)
    return (src.reshape(2, N_TILES, N_CHUNKS, CHUNK),
            dst.reshape(2, N_TILES, N_CHUNKS, CHUNK))


def kernel(x,
           edge_index_l1_r0, edge_index_l1_r1,
           edge_index_l2_r0, edge_index_l2_r1,
           edge_index_l3_r0, edge_index_l3_r1,
           W1_r0, b1_r0, W1_r1, b1_r1,
           W2_r0, b2_r0, W2_r1, b2_r1,
           W3_r0, b3_r0, W3_r1, b3_r1):
    s1, d1 = _prep_edges(edge_index_l1_r0, edge_index_l1_r1)
    s2, d2 = _prep_edges(edge_index_l2_r0, edge_index_l2_r1)
    s3, d3 = _prep_edges(edge_index_l3_r0, edge_index_l3_r1)

    x_aug = jnp.concatenate(
        [x, jnp.ones((N, 1), jnp.float32), jnp.zeros((N, DA - D - 1),
                                                     jnp.float32)], axis=1)

    acc = _SC_SEG_SUM(x_aug, s1, d1)
    _, h1 = _tc_combine(acc, W1_r0, W1_r1, (b1_r0 + b1_r1).reshape(1, D), True)
    acc = _SC_SEG_SUM(h1, s2, d2)
    x1, h2 = _tc_combine(acc, W2_r0, W2_r1, (b2_r0 + b2_r1).reshape(1, D), True)
    acc = _SC_SEG_SUM(h2, s3, d3)
    _, h3 = _tc_combine(acc, W3_r0, W3_r1, (b3_r0 + b3_r1).reshape(1, D), False)
    return h3, x1
